# baseline (speedup 1.0000x reference)
import functools

import jax
import jax.numpy as jnp
from jax import lax
from jax.experimental import pallas as pl
from jax.experimental.pallas import tpu as pltpu
from jax.experimental.pallas import tpu_sc as plsc

VOCAB_ROWS = 1000000
D = 64
B_TOTAL = 4096 * 200
CHUNK = 128
NUM_CHUNKS = B_TOTAL // CHUNK
SCALE = float(D) ** -0.5


def _make_sc_kernel():
    info = plsc.get_sparse_core_info()
    nc, ns = info.num_cores, info.num_subcores
    nw = nc * ns
    chunks_per_w = NUM_CHUNKS // nw

    mesh = plsc.VectorSubcoreMesh(core_axis_name="c", subcore_axis_name="s")

    @functools.partial(
        pl.kernel,
        out_type=jax.ShapeDtypeStruct((B_TOTAL, D), jnp.float32),
        mesh=mesh,
        scratch_types=[
            pltpu.VMEM((chunks_per_w, CHUNK), jnp.int32),
            pltpu.VMEM((CHUNK, D), jnp.float32),
            pltpu.SemaphoreType.DMA,
        ],
        compiler_params=pltpu.CompilerParams(use_tc_tiling_on_sc=False),
    )
    def emb_kernel(idx_hbm, table_hbm, out_hbm, idx_v, rows_v, sem):
        wid = lax.axis_index("s") * nc + lax.axis_index("c")
        chunk_base = wid * chunks_per_w
        pltpu.sync_copy(idx_hbm.at[pl.ds(chunk_base, chunks_per_w)], idx_v)

        def chunk_body(c, carry):
            pltpu.async_copy(table_hbm.at[idx_v.at[c]], rows_v, sem).wait()

            def scale_row(i, carry2):
                for j in range(D // 16):
                    s = pl.ds(j * 16, 16)
                    rows_v[i, s] = rows_v[i, s] * SCALE
                return carry2

            lax.fori_loop(0, CHUNK, scale_row, 0, unroll=4)
            out_start = (chunk_base + c) * CHUNK
            pltpu.sync_copy(rows_v, out_hbm.at[pl.ds(out_start, CHUNK)])
            return carry

        lax.fori_loop(0, chunks_per_w, chunk_body, 0)

    return emb_kernel


_emb = _make_sc_kernel()


@jax.jit
def kernel(x, table):
    idx = x.reshape(NUM_CHUNKS, CHUNK).astype(jnp.int32)
    out = _emb(idx, table)
    return out.reshape(x.shape + (D,))


# speedup vs baseline: 1.0563x; 1.0563x over previous
import functools

import jax
import jax.numpy as jnp
from jax import lax
from jax.experimental import pallas as pl
from jax.experimental.pallas import tpu as pltpu
from jax.experimental.pallas import tpu_sc as plsc

D = 64
B_TOTAL = 4096 * 200
CHUNK = 128
NUM_CHUNKS = B_TOTAL // CHUNK
SCALE = float(D) ** -0.5
NBUF = 4


def _make_sc_kernel():
    info = plsc.get_sparse_core_info()
    nc, ns = info.num_cores, info.num_subcores
    nw = nc * ns
    cpw = NUM_CHUNKS // nw
    nsteps = cpw // NBUF

    mesh = plsc.VectorSubcoreMesh(core_axis_name="c", subcore_axis_name="s")

    scratch = [
        pltpu.VMEM((cpw, CHUNK), jnp.int32),
        pltpu.VMEM((NBUF, CHUNK, D), jnp.float32),
        pltpu.VMEM((NBUF, CHUNK, D), jnp.float32),
    ] + [pltpu.SemaphoreType.DMA] * (2 * NBUF)

    @functools.partial(
        pl.kernel,
        out_type=jax.ShapeDtypeStruct((B_TOTAL, D), jnp.float32),
        mesh=mesh,
        scratch_types=scratch,
        compiler_params=pltpu.CompilerParams(use_tc_tiling_on_sc=False),
    )
    def emb_kernel(idx_hbm, table_hbm, out_hbm, idx_v, in_v, sc_v, *sems):
        gsem = sems[:NBUF]
        ssem = sems[NBUF:]
        wid = lax.axis_index("s") * nc + lax.axis_index("c")
        chunk_base = wid * cpw
        pltpu.sync_copy(idx_hbm.at[pl.ds(chunk_base, cpw)], idx_v)

        def start_gather(b, c):
            pltpu.async_copy(table_hbm.at[idx_v.at[c]], in_v.at[b], gsem[b])

        def start_store(b, c):
            out_start = (chunk_base + c) * CHUNK
            pltpu.async_copy(sc_v.at[b], out_hbm.at[pl.ds(out_start, CHUNK)],
                             ssem[b])

        def wait_gather(b):
            pltpu.make_async_copy(table_hbm.at[idx_v.at[0]], in_v.at[b],
                                  gsem[b]).wait()

        def wait_store(b):
            pltpu.make_async_copy(sc_v.at[b], out_hbm.at[pl.ds(0, CHUNK)],
                                  ssem[b]).wait()

        def scale(b):
            def row(i, carry):
                for j in range(D // 16):
                    s = pl.ds(j * 16, 16)
                    sc_v[b, i, s] = in_v[b, i, s] * SCALE
                return carry

            lax.fori_loop(0, CHUNK, row, 0, unroll=4)

        for b in range(NBUF):
            start_gather(b, b)
        for b in range(NBUF):
            wait_gather(b)
            scale(b)
            start_store(b, b)
            start_gather(b, NBUF + b)

        def step(g0, carry):
            for b in range(NBUF):
                c = g0 * NBUF + b
                wait_gather(b)
                wait_store(b)
                scale(b)
                start_store(b, c)
                start_gather(b, c + NBUF)
            return carry

        lax.fori_loop(1, nsteps - 1, step, 0)

        for b in range(NBUF):
            c = (nsteps - 1) * NBUF + b
            wait_gather(b)
            wait_store(b)
            scale(b)
            start_store(b, c)
        for b in range(NBUF):
            wait_store(b)

    return emb_kernel


_emb = _make_sc_kernel()


@jax.jit
def kernel(x, table):
    idx = x.reshape(NUM_CHUNKS, CHUNK).astype(jnp.int32)
    out = _emb(idx, table)
    return out.reshape(x.shape + (D,))
